# baseline (speedup 1.0000x reference)
import functools

import jax
import jax.numpy as jnp
from jax import lax
from jax.experimental import pallas as pl
from jax.experimental.pallas import tpu as pltpu
from jax.experimental.pallas import tpu_sc as plsc

N = 10000
E = 320000
D = 128
G = 64
OUT = 10

NC = 2
NS = 16
NW = NC * NS
CHUNK = 128
NCHUNK = 80
EDGES_PER_TILE = NCHUNK * CHUNK
EPAD = NW * EDGES_PER_TILE
NPAD = 10240
ROWS_PER_TILE = NPAD // NS

BR = 1000
NBLK = N // BR


@functools.partial(
    pl.kernel,
    out_type=jax.ShapeDtypeStruct((NC, NPAD, D), jnp.float32),
    mesh=plsc.VectorSubcoreMesh(core_axis_name="c", subcore_axis_name="s"),
    scratch_types=[
        pltpu.VMEM((CHUNK,), jnp.int32),
        pltpu.VMEM((CHUNK,), jnp.int32),
        pltpu.VMEM((CHUNK,), jnp.int32),
        pltpu.VMEM((CHUNK,), jnp.int32),
        pltpu.VMEM((CHUNK, D), jnp.float32),
        pltpu.VMEM((CHUNK, D), jnp.float32),
        pltpu.VMEM_SHARED((NPAD, D), jnp.float32),
        pltpu.SemaphoreType.DMA,
        pltpu.SemaphoreType.DMA,
        pltpu.SemaphoreType.DMA,
        pltpu.SemaphoreType.DMA,
    ],
)
def _segsum_sc(x_hbm, src_hbm, dst_hbm, zeros_hbm, out_hbm,
               src_a, dst_a, src_b, dst_b, rows_a, rows_b, acc_sh,
               sem_a, sem_b, sem_ia, sem_ib):
    c = lax.axis_index("c")
    s = lax.axis_index("s")
    wid = c * NS + s

    pltpu.async_copy(src_hbm.at[wid, 0], src_a, sem_ia)
    pltpu.async_copy(dst_hbm.at[wid, 0], dst_a, sem_ia)
    pltpu.async_copy(src_hbm.at[wid, 1], src_b, sem_ib)
    pltpu.async_copy(dst_hbm.at[wid, 1], dst_b, sem_ib)
    pltpu.sync_copy(zeros_hbm.at[pl.ds(s * ROWS_PER_TILE, ROWS_PER_TILE)],
                    acc_sh.at[pl.ds(s * ROWS_PER_TILE, ROWS_PER_TILE)])
    plsc.subcore_barrier()

    def _wait_idx(sem):
        pltpu.make_async_copy(src_hbm.at[wid, 0], src_a, sem).wait()
        pltpu.make_async_copy(dst_hbm.at[wid, 0], dst_a, sem).wait()

    def _wait_rows(rows, sem):
        pltpu.make_async_copy(x_hbm.at[src_a], rows, sem).wait()

    _wait_idx(sem_ia)
    pltpu.async_copy(x_hbm.at[src_a], rows_a, sem_a)
    _wait_idx(sem_ib)
    pltpu.async_copy(x_hbm.at[src_b], rows_b, sem_b)

    def body(j, carry):
        i = 2 * j
        _wait_rows(rows_a, sem_a)
        pltpu.sync_copy(rows_a, acc_sh.at[dst_a], add=True)
        pltpu.async_copy(src_hbm.at[wid, i + 2], src_a, sem_ia)
        pltpu.async_copy(dst_hbm.at[wid, i + 2], dst_a, sem_ia)
        _wait_rows(rows_b, sem_b)
        pltpu.sync_copy(rows_b, acc_sh.at[dst_b], add=True)
        pltpu.async_copy(src_hbm.at[wid, i + 3], src_b, sem_ib)
        pltpu.async_copy(dst_hbm.at[wid, i + 3], dst_b, sem_ib)
        _wait_idx(sem_ia)
        pltpu.async_copy(x_hbm.at[src_a], rows_a, sem_a)
        _wait_idx(sem_ib)
        pltpu.async_copy(x_hbm.at[src_b], rows_b, sem_b)
        return carry

    lax.fori_loop(0, NCHUNK // 2 - 1, body, 0)
    _wait_rows(rows_a, sem_a)
    pltpu.sync_copy(rows_a, acc_sh.at[dst_a], add=True)
    _wait_rows(rows_b, sem_b)
    pltpu.sync_copy(rows_b, acc_sh.at[dst_b], add=True)
    plsc.subcore_barrier()

    pltpu.sync_copy(acc_sh.at[pl.ds(s * ROWS_PER_TILE, ROWS_PER_TILE)],
                    out_hbm.at[c, pl.ds(s * ROWS_PER_TILE, ROWS_PER_TILE)])


def _layer_body(parts_ref, x_ref, wrel_ref, brel_ref, wroot_ref, o_ref):
    agg = parts_ref[0] + parts_ref[1]
    acc = jnp.dot(agg, wrel_ref[...], preferred_element_type=jnp.float32)
    acc += jnp.dot(x_ref[...], wroot_ref[...], preferred_element_type=jnp.float32)
    acc += brel_ref[...]
    o_ref[...] = jnp.maximum(acc, 0.0)


def _layer_tc(parts, x, wrel, brel, wroot):
    return pl.pallas_call(
        _layer_body,
        grid=(NBLK,),
        in_specs=[
            pl.BlockSpec((NC, BR, D), lambda i: (0, i, 0)),
            pl.BlockSpec((BR, D), lambda i: (i, 0)),
            pl.BlockSpec((D, D), lambda i: (0, 0)),
            pl.BlockSpec((1, D), lambda i: (0, 0)),
            pl.BlockSpec((D, D), lambda i: (0, 0)),
        ],
        out_specs=pl.BlockSpec((BR, D), lambda i: (i, 0)),
        out_shape=jax.ShapeDtypeStruct((N, D), jnp.float32),
    )(parts, x, wrel, brel.reshape(1, D), wroot)


def _final_body(parts_ref, x_ref, wrel_ref, brel_ref, wroot_ref,
                wlin1_ref, blin1_ref, wlin2_ref, blin2_ref, batch_ref, o_ref):
    agg = parts_ref[0] + parts_ref[1]
    h = jnp.dot(agg, wrel_ref[...], preferred_element_type=jnp.float32)
    h += jnp.dot(x_ref[...], wroot_ref[...], preferred_element_type=jnp.float32)
    h += brel_ref[...]
    h = jnp.maximum(h, 0.0)
    h = jnp.maximum(
        jnp.dot(h, wlin1_ref[...], preferred_element_type=jnp.float32)
        + blin1_ref[...], 0.0)
    y = jnp.dot(h, wlin2_ref[...], preferred_element_type=jnp.float32)
    y += blin2_ref[...]
    seg = lax.broadcasted_iota(jnp.int32, (BR, G), 1)
    mask = (batch_ref[...] == seg).astype(jnp.float32)
    contrib = lax.dot_general(mask, y, (((0,), (0,)), ((), ())),
                              preferred_element_type=jnp.float32)

    @pl.when(pl.program_id(0) == 0)
    def _():
        o_ref[...] = jnp.zeros_like(o_ref)

    o_ref[...] += contrib


def _final_tc(parts, x, wrel, brel, wroot, wlin1, blin1, wlin2, blin2, batch):
    return pl.pallas_call(
        _final_body,
        grid=(NBLK,),
        in_specs=[
            pl.BlockSpec((NC, BR, D), lambda i: (0, i, 0)),
            pl.BlockSpec((BR, D), lambda i: (i, 0)),
            pl.BlockSpec((D, D), lambda i: (0, 0)),
            pl.BlockSpec((1, D), lambda i: (0, 0)),
            pl.BlockSpec((D, D), lambda i: (0, 0)),
            pl.BlockSpec((D, D), lambda i: (0, 0)),
            pl.BlockSpec((1, D), lambda i: (0, 0)),
            pl.BlockSpec((D, OUT), lambda i: (0, 0)),
            pl.BlockSpec((1, OUT), lambda i: (0, 0)),
            pl.BlockSpec((BR, 1), lambda i: (i, 0)),
        ],
        out_specs=pl.BlockSpec((G, OUT), lambda i: (0, 0)),
        out_shape=jax.ShapeDtypeStruct((G, OUT), jnp.float32),
    )(parts, x, wrel, brel.reshape(1, D), wroot,
      wlin1, blin1.reshape(1, D), wlin2, blin2.reshape(1, OUT),
      batch.reshape(N, 1))


def kernel(x, edge_index, batch,
           Wrel0, brel0, Wroot0,
           Wrel1, brel1, Wroot1,
           Wrel2, brel2, Wroot2,
           Wlin1, blin1, Wlin2, blin2):
    npad_e = EPAD - E
    src = jnp.concatenate(
        [edge_index[0], jnp.zeros((npad_e,), jnp.int32)]).reshape(NW, NCHUNK, CHUNK)
    dst = jnp.concatenate(
        [edge_index[1], N + (jnp.arange(npad_e, dtype=jnp.int32) % (NPAD - N))]
    ).reshape(NW, NCHUNK, CHUNK)
    zeros = jnp.zeros((NPAD, D), jnp.float32)

    parts = _segsum_sc(x, src, dst, zeros)
    h = _layer_tc(parts, x, Wrel0, brel0, Wroot0)
    parts = _segsum_sc(h, src, dst, zeros)
    h = _layer_tc(parts, h, Wrel1, brel1, Wroot1)
    parts = _segsum_sc(h, src, dst, zeros)
    return _final_tc(parts, h, Wrel2, brel2, Wroot2,
                     Wlin1, blin1, Wlin2, blin2, batch)


# speedup vs baseline: 3.9845x; 3.9845x over previous
import functools

import jax
import jax.numpy as jnp
from jax import lax
from jax.experimental import pallas as pl
from jax.experimental.pallas import tpu as pltpu
from jax.experimental.pallas import tpu_sc as plsc

N = 10000
E = 320000
D = 128
G = 64
OUT = 10

NC = 2
NS = 16
NW = NC * NS
CHUNK = 128
NCHUNK = 80
EDGES_PER_TILE = NCHUNK * CHUNK
EPAD = NW * EDGES_PER_TILE
NPAD = 10240
ROWS_PER_TILE = NPAD // NS

BR = 1000
NBLK = N // BR


@functools.partial(
    pl.kernel,
    out_type=jax.ShapeDtypeStruct((NC, NPAD, D), jnp.float32),
    mesh=plsc.VectorSubcoreMesh(core_axis_name="c", subcore_axis_name="s"),
    scratch_types=[
        pltpu.VMEM((NCHUNK, CHUNK), jnp.int32),
        pltpu.VMEM((CHUNK,), jnp.int32),
        pltpu.VMEM((CHUNK,), jnp.int32),
        pltpu.VMEM((CHUNK,), jnp.int32),
        pltpu.VMEM((CHUNK,), jnp.int32),
        pltpu.VMEM((CHUNK, D), jnp.float32),
        pltpu.VMEM((CHUNK, D), jnp.float32),
        pltpu.VMEM_SHARED((NPAD, D), jnp.float32),
        pltpu.SemaphoreType.DMA,
        pltpu.SemaphoreType.DMA,
        pltpu.SemaphoreType.DMA,
        pltpu.SemaphoreType.DMA,
        pltpu.SemaphoreType.DMA,
        pltpu.SemaphoreType.DMA,
    ],
)
def _segsum_sc(x_hbm, src_hbm, dst_hbm, zeros_hbm, out_hbm,
               dst_v, sr0, sr1, sr2, sr3, rows_a, rows_b, acc_sh,
               sem_a, sem_b, ss0, ss1, ss2, ss3):
    c = lax.axis_index("c")
    s = lax.axis_index("s")
    wid = c * NS + s
    srcb = (sr0, sr1, sr2, sr3)
    ssem = (ss0, ss1, ss2, ss3)
    rows = (rows_a, rows_b)
    rsem = (sem_a, sem_b)

    for k in range(4):
        pltpu.async_copy(src_hbm.at[wid, k], srcb[k], ssem[k])
    pltpu.sync_copy(dst_hbm.at[wid], dst_v)
    pltpu.sync_copy(zeros_hbm.at[pl.ds(s * ROWS_PER_TILE, ROWS_PER_TILE)],
                    acc_sh.at[pl.ds(s * ROWS_PER_TILE, ROWS_PER_TILE)])
    plsc.subcore_barrier()

    def _wait_src(k):
        pltpu.make_async_copy(src_hbm.at[wid, 0], srcb[k], ssem[k]).wait()

    def _wait_rows(k):
        pltpu.make_async_copy(x_hbm.at[sr0], rows[k], rsem[k]).wait()

    _wait_src(0)
    pltpu.async_copy(x_hbm.at[sr0], rows_a, sem_a)
    _wait_src(1)
    pltpu.async_copy(x_hbm.at[sr1], rows_b, sem_b)

    def _step(i, k, prefetch):
        _wait_rows(k % 2)
        pltpu.sync_copy(rows[k % 2], acc_sh.at[dst_v.at[i]], add=True)
        if prefetch:
            _wait_src((k + 2) % 4)
            pltpu.async_copy(x_hbm.at[srcb[(k + 2) % 4]], rows[k % 2],
                             rsem[k % 2])
            pltpu.async_copy(src_hbm.at[wid, i + 4], srcb[k], ssem[k])

    def body(j, carry):
        i = 4 * j
        for k in range(4):
            _step(i + k, k, True)
        return carry

    lax.fori_loop(0, NCHUNK // 4 - 1, body, 0)
    base = NCHUNK - 4
    for k in range(4):
        _wait_rows(k % 2)
        pltpu.sync_copy(rows[k % 2], acc_sh.at[dst_v.at[base + k]], add=True)
        if k < 2:
            _wait_src((k + 2) % 4)
            pltpu.async_copy(x_hbm.at[srcb[(k + 2) % 4]], rows[k % 2],
                             rsem[k % 2])
    plsc.subcore_barrier()

    pltpu.sync_copy(acc_sh.at[pl.ds(s * ROWS_PER_TILE, ROWS_PER_TILE)],
                    out_hbm.at[c, pl.ds(s * ROWS_PER_TILE, ROWS_PER_TILE)])


def _layer_body(parts_ref, x_ref, wrel_ref, brel_ref, wroot_ref, o_ref):
    agg = parts_ref[0] + parts_ref[1]
    acc = jnp.dot(agg, wrel_ref[...], preferred_element_type=jnp.float32)
    acc += jnp.dot(x_ref[...], wroot_ref[...], preferred_element_type=jnp.float32)
    acc += brel_ref[...]
    o_ref[...] = jnp.maximum(acc, 0.0)


def _layer_tc(parts, x, wrel, brel, wroot):
    return pl.pallas_call(
        _layer_body,
        grid=(NBLK,),
        in_specs=[
            pl.BlockSpec((NC, BR, D), lambda i: (0, i, 0)),
            pl.BlockSpec((BR, D), lambda i: (i, 0)),
            pl.BlockSpec((D, D), lambda i: (0, 0)),
            pl.BlockSpec((1, D), lambda i: (0, 0)),
            pl.BlockSpec((D, D), lambda i: (0, 0)),
        ],
        out_specs=pl.BlockSpec((BR, D), lambda i: (i, 0)),
        out_shape=jax.ShapeDtypeStruct((N, D), jnp.float32),
    )(parts, x, wrel, brel.reshape(1, D), wroot)


def _final_body(parts_ref, x_ref, wrel_ref, brel_ref, wroot_ref,
                wlin1_ref, blin1_ref, wlin2_ref, blin2_ref, batch_ref, o_ref):
    agg = parts_ref[0] + parts_ref[1]
    h = jnp.dot(agg, wrel_ref[...], preferred_element_type=jnp.float32)
    h += jnp.dot(x_ref[...], wroot_ref[...], preferred_element_type=jnp.float32)
    h += brel_ref[...]
    h = jnp.maximum(h, 0.0)
    h = jnp.maximum(
        jnp.dot(h, wlin1_ref[...], preferred_element_type=jnp.float32)
        + blin1_ref[...], 0.0)
    y = jnp.dot(h, wlin2_ref[...], preferred_element_type=jnp.float32)
    y += blin2_ref[...]
    seg = lax.broadcasted_iota(jnp.int32, (BR, G), 1)
    mask = (batch_ref[...] == seg).astype(jnp.float32)
    contrib = lax.dot_general(mask, y, (((0,), (0,)), ((), ())),
                              preferred_element_type=jnp.float32)

    @pl.when(pl.program_id(0) == 0)
    def _():
        o_ref[...] = jnp.zeros_like(o_ref)

    o_ref[...] += contrib


def _final_tc(parts, x, wrel, brel, wroot, wlin1, blin1, wlin2, blin2, batch):
    return pl.pallas_call(
        _final_body,
        grid=(NBLK,),
        in_specs=[
            pl.BlockSpec((NC, BR, D), lambda i: (0, i, 0)),
            pl.BlockSpec((BR, D), lambda i: (i, 0)),
            pl.BlockSpec((D, D), lambda i: (0, 0)),
            pl.BlockSpec((1, D), lambda i: (0, 0)),
            pl.BlockSpec((D, D), lambda i: (0, 0)),
            pl.BlockSpec((D, D), lambda i: (0, 0)),
            pl.BlockSpec((1, D), lambda i: (0, 0)),
            pl.BlockSpec((D, OUT), lambda i: (0, 0)),
            pl.BlockSpec((1, OUT), lambda i: (0, 0)),
            pl.BlockSpec((BR, 1), lambda i: (i, 0)),
        ],
        out_specs=pl.BlockSpec((G, OUT), lambda i: (0, 0)),
        out_shape=jax.ShapeDtypeStruct((G, OUT), jnp.float32),
    )(parts, x, wrel, brel.reshape(1, D), wroot,
      wlin1, blin1.reshape(1, D), wlin2, blin2.reshape(1, OUT),
      batch.reshape(N, 1))


def kernel(x, edge_index, batch,
           Wrel0, brel0, Wroot0,
           Wrel1, brel1, Wroot1,
           Wrel2, brel2, Wroot2,
           Wlin1, blin1, Wlin2, blin2):
    npad_e = EPAD - E
    src = jnp.concatenate(
        [edge_index[0], jnp.arange(npad_e, dtype=jnp.int32) % N]
    ).reshape(NW, NCHUNK, CHUNK)
    dst = jnp.concatenate(
        [edge_index[1], N + (jnp.arange(npad_e, dtype=jnp.int32) % (NPAD - N))]
    ).reshape(NW, NCHUNK, CHUNK)
    zeros = jnp.zeros((NPAD, D), jnp.float32)

    parts = _segsum_sc(x, src, dst, zeros)
    h = _layer_tc(parts, x, Wrel0, brel0, Wroot0)
    parts = _segsum_sc(h, src, dst, zeros)
    h = _layer_tc(parts, h, Wrel1, brel1, Wroot1)
    parts = _segsum_sc(h, src, dst, zeros)
    return _final_tc(parts, h, Wrel2, brel2, Wroot2,
                     Wlin1, blin1, Wlin2, blin2, batch)


# speedup vs baseline: 4.2749x; 1.0729x over previous
import functools

import jax
import jax.numpy as jnp
from jax import lax
from jax.experimental import pallas as pl
from jax.experimental.pallas import tpu as pltpu
from jax.experimental.pallas import tpu_sc as plsc

N = 10000
E = 320000
D = 128
G = 64
OUT = 10

NC = 2
NS = 16
NW = NC * NS
CHUNK = 80
NCHUNK = 126
EDGES_PER_TILE = NCHUNK * CHUNK
EPAD = NW * EDGES_PER_TILE
NPAD = 10240
ROWS_PER_TILE = NPAD // NS

BR = 1000
NBLK = N // BR


@functools.partial(
    pl.kernel,
    out_type=jax.ShapeDtypeStruct((NC, NPAD, D), jnp.float32),
    mesh=plsc.VectorSubcoreMesh(core_axis_name="c", subcore_axis_name="s"),
    scratch_types=[
        pltpu.VMEM((NCHUNK, CHUNK), jnp.int32),
        pltpu.VMEM((CHUNK,), jnp.int32),
        pltpu.VMEM((CHUNK,), jnp.int32),
        pltpu.VMEM((CHUNK,), jnp.int32),
        pltpu.VMEM((CHUNK,), jnp.int32),
        pltpu.VMEM((CHUNK,), jnp.int32),
        pltpu.VMEM((CHUNK,), jnp.int32),
        pltpu.VMEM((CHUNK, D), jnp.float32),
        pltpu.VMEM((CHUNK, D), jnp.float32),
        pltpu.VMEM((CHUNK, D), jnp.float32),
        pltpu.VMEM_SHARED((NPAD, D), jnp.float32),
        pltpu.SemaphoreType.DMA,
        pltpu.SemaphoreType.DMA,
        pltpu.SemaphoreType.DMA,
        pltpu.SemaphoreType.DMA,
        pltpu.SemaphoreType.DMA,
        pltpu.SemaphoreType.DMA,
        pltpu.SemaphoreType.DMA,
        pltpu.SemaphoreType.DMA,
        pltpu.SemaphoreType.DMA,
    ],
)
def _segsum_sc(x_hbm, src_hbm, dst_hbm, zeros_hbm, out_hbm,
               dst_v, sr0, sr1, sr2, sr3, sr4, sr5,
               rows_a, rows_b, rows_c, acc_sh,
               sem_a, sem_b, sem_c, ss0, ss1, ss2, ss3, ss4, ss5):
    c = lax.axis_index("c")
    s = lax.axis_index("s")
    wid = c * NS + s
    srcb = (sr0, sr1, sr2, sr3, sr4, sr5)
    ssem = (ss0, ss1, ss2, ss3, ss4, ss5)
    rows = (rows_a, rows_b, rows_c)
    rsem = (sem_a, sem_b, sem_c)

    for k in range(6):
        pltpu.async_copy(src_hbm.at[wid, k], srcb[k], ssem[k])
    pltpu.sync_copy(dst_hbm.at[wid], dst_v)
    pltpu.sync_copy(zeros_hbm.at[pl.ds(s * ROWS_PER_TILE, ROWS_PER_TILE)],
                    acc_sh.at[pl.ds(s * ROWS_PER_TILE, ROWS_PER_TILE)])
    plsc.subcore_barrier()

    def _wait_src(k):
        pltpu.make_async_copy(src_hbm.at[wid, 0], srcb[k], ssem[k]).wait()

    def _wait_rows(k):
        pltpu.make_async_copy(x_hbm.at[sr0], rows[k], rsem[k]).wait()

    for k in range(3):
        _wait_src(k)
        pltpu.async_copy(x_hbm.at[srcb[k]], rows[k], rsem[k])

    def _step(i, k, prefetch):
        _wait_rows(k % 3)
        pltpu.sync_copy(rows[k % 3], acc_sh.at[dst_v.at[i]], add=True)
        if prefetch:
            _wait_src((k + 3) % 6)
            pltpu.async_copy(x_hbm.at[srcb[(k + 3) % 6]], rows[k % 3],
                             rsem[k % 3])
            pltpu.async_copy(src_hbm.at[wid, i + 6], srcb[k % 6], ssem[k % 6])

    def body(j, carry):
        i = 6 * j
        for k in range(6):
            _step(i + k, k, True)
        return carry

    lax.fori_loop(0, NCHUNK // 6 - 1, body, 0)
    base = NCHUNK - 6
    for k in range(6):
        _wait_rows(k % 3)
        pltpu.sync_copy(rows[k % 3], acc_sh.at[dst_v.at[base + k]], add=True)
        if k < 3:
            _wait_src((k + 3) % 6)
            pltpu.async_copy(x_hbm.at[srcb[(k + 3) % 6]], rows[k % 3],
                             rsem[k % 3])
    plsc.subcore_barrier()

    pltpu.sync_copy(acc_sh.at[pl.ds(s * ROWS_PER_TILE, ROWS_PER_TILE)],
                    out_hbm.at[c, pl.ds(s * ROWS_PER_TILE, ROWS_PER_TILE)])


def _layer_body(parts_ref, x_ref, wrel_ref, brel_ref, wroot_ref, o_ref):
    agg = parts_ref[0] + parts_ref[1]
    acc = jnp.dot(agg, wrel_ref[...], preferred_element_type=jnp.float32)
    acc += jnp.dot(x_ref[...], wroot_ref[...], preferred_element_type=jnp.float32)
    acc += brel_ref[...]
    o_ref[...] = jnp.maximum(acc, 0.0)


def _layer_tc(parts, x, wrel, brel, wroot):
    return pl.pallas_call(
        _layer_body,
        grid=(NBLK,),
        in_specs=[
            pl.BlockSpec((NC, BR, D), lambda i: (0, i, 0)),
            pl.BlockSpec((BR, D), lambda i: (i, 0)),
            pl.BlockSpec((D, D), lambda i: (0, 0)),
            pl.BlockSpec((1, D), lambda i: (0, 0)),
            pl.BlockSpec((D, D), lambda i: (0, 0)),
        ],
        out_specs=pl.BlockSpec((BR, D), lambda i: (i, 0)),
        out_shape=jax.ShapeDtypeStruct((N, D), jnp.float32),
    )(parts, x, wrel, brel.reshape(1, D), wroot)


def _final_body(parts_ref, x_ref, wrel_ref, brel_ref, wroot_ref,
                wlin1_ref, blin1_ref, wlin2_ref, blin2_ref, batch_ref, o_ref):
    agg = parts_ref[0] + parts_ref[1]
    h = jnp.dot(agg, wrel_ref[...], preferred_element_type=jnp.float32)
    h += jnp.dot(x_ref[...], wroot_ref[...], preferred_element_type=jnp.float32)
    h += brel_ref[...]
    h = jnp.maximum(h, 0.0)
    h = jnp.maximum(
        jnp.dot(h, wlin1_ref[...], preferred_element_type=jnp.float32)
        + blin1_ref[...], 0.0)
    y = jnp.dot(h, wlin2_ref[...], preferred_element_type=jnp.float32)
    y += blin2_ref[...]
    seg = lax.broadcasted_iota(jnp.int32, (BR, G), 1)
    mask = (batch_ref[...] == seg).astype(jnp.float32)
    contrib = lax.dot_general(mask, y, (((0,), (0,)), ((), ())),
                              preferred_element_type=jnp.float32)

    @pl.when(pl.program_id(0) == 0)
    def _():
        o_ref[...] = jnp.zeros_like(o_ref)

    o_ref[...] += contrib


def _final_tc(parts, x, wrel, brel, wroot, wlin1, blin1, wlin2, blin2, batch):
    return pl.pallas_call(
        _final_body,
        grid=(NBLK,),
        in_specs=[
            pl.BlockSpec((NC, BR, D), lambda i: (0, i, 0)),
            pl.BlockSpec((BR, D), lambda i: (i, 0)),
            pl.BlockSpec((D, D), lambda i: (0, 0)),
            pl.BlockSpec((1, D), lambda i: (0, 0)),
            pl.BlockSpec((D, D), lambda i: (0, 0)),
            pl.BlockSpec((D, D), lambda i: (0, 0)),
            pl.BlockSpec((1, D), lambda i: (0, 0)),
            pl.BlockSpec((D, OUT), lambda i: (0, 0)),
            pl.BlockSpec((1, OUT), lambda i: (0, 0)),
            pl.BlockSpec((BR, 1), lambda i: (i, 0)),
        ],
        out_specs=pl.BlockSpec((G, OUT), lambda i: (0, 0)),
        out_shape=jax.ShapeDtypeStruct((G, OUT), jnp.float32),
    )(parts, x, wrel, brel.reshape(1, D), wroot,
      wlin1, blin1.reshape(1, D), wlin2, blin2.reshape(1, OUT),
      batch.reshape(N, 1))


def kernel(x, edge_index, batch,
           Wrel0, brel0, Wroot0,
           Wrel1, brel1, Wroot1,
           Wrel2, brel2, Wroot2,
           Wlin1, blin1, Wlin2, blin2):
    npad_e = EPAD - E
    src = jnp.concatenate(
        [edge_index[0], jnp.arange(npad_e, dtype=jnp.int32) % N]
    ).reshape(NW, NCHUNK, CHUNK)
    dst = jnp.concatenate(
        [edge_index[1], N + (jnp.arange(npad_e, dtype=jnp.int32) % (NPAD - N))]
    ).reshape(NW, NCHUNK, CHUNK)
    zeros = jnp.zeros((NPAD, D), jnp.float32)

    parts = _segsum_sc(x, src, dst, zeros)
    h = _layer_tc(parts, x, Wrel0, brel0, Wroot0)
    parts = _segsum_sc(h, src, dst, zeros)
    h = _layer_tc(parts, h, Wrel1, brel1, Wroot1)
    parts = _segsum_sc(h, src, dst, zeros)
    return _final_tc(parts, h, Wrel2, brel2, Wroot2,
                     Wlin1, blin1, Wlin2, blin2, batch)


# speedup vs baseline: 4.4797x; 1.0479x over previous
import functools

import jax
import jax.numpy as jnp
from jax import lax
from jax.experimental import pallas as pl
from jax.experimental.pallas import tpu as pltpu
from jax.experimental.pallas import tpu_sc as plsc

N = 10000
E = 320000
D = 128
G = 64
OUT = 10

NC = 2
NS = 16
NW = NC * NS
CHUNK = 80
NCHUNK = 126
EDGES_PER_TILE = NCHUNK * CHUNK
EPAD = NW * EDGES_PER_TILE
NPAD = 10240
ROWS_PER_TILE = NPAD // NS

BR = 2000
NBLK = N // BR


@functools.partial(
    pl.kernel,
    out_type=jax.ShapeDtypeStruct((NC, NPAD, D), jnp.float32),
    mesh=plsc.VectorSubcoreMesh(core_axis_name="c", subcore_axis_name="s"),
    scratch_types=[
        pltpu.VMEM((NCHUNK, CHUNK), jnp.int32),
        pltpu.VMEM((CHUNK,), jnp.int32),
        pltpu.VMEM((CHUNK,), jnp.int32),
        pltpu.VMEM((CHUNK,), jnp.int32),
        pltpu.VMEM((CHUNK,), jnp.int32),
        pltpu.VMEM((CHUNK,), jnp.int32),
        pltpu.VMEM((CHUNK,), jnp.int32),
        pltpu.VMEM((CHUNK, D), jnp.float32),
        pltpu.VMEM((CHUNK, D), jnp.float32),
        pltpu.VMEM((CHUNK, D), jnp.float32),
        pltpu.VMEM_SHARED((NPAD, D), jnp.float32),
        pltpu.SemaphoreType.DMA,
        pltpu.SemaphoreType.DMA,
        pltpu.SemaphoreType.DMA,
        pltpu.SemaphoreType.DMA,
        pltpu.SemaphoreType.DMA,
        pltpu.SemaphoreType.DMA,
        pltpu.SemaphoreType.DMA,
        pltpu.SemaphoreType.DMA,
        pltpu.SemaphoreType.DMA,
        pltpu.SemaphoreType.DMA,
    ],
)
def _segsum_sc(x_hbm, src_hbm, dst_hbm, zeros_hbm, out_hbm,
               dst_v, sr0, sr1, sr2, sr3, sr4, sr5,
               rows_a, rows_b, rows_c, acc_sh,
               sem_a, sem_b, sem_c, ss0, ss1, ss2, ss3, ss4, ss5, sem_d):
    c = lax.axis_index("c")
    s = lax.axis_index("s")
    wid = c * NS + s
    srcb = (sr0, sr1, sr2, sr3, sr4, sr5)
    ssem = (ss0, ss1, ss2, ss3, ss4, ss5)
    rows = (rows_a, rows_b, rows_c)
    rsem = (sem_a, sem_b, sem_c)

    def _wait_src(k):
        pltpu.make_async_copy(src_hbm.at[wid, 0], srcb[k], ssem[k]).wait()

    def _wait_rows(k):
        pltpu.make_async_copy(x_hbm.at[sr0], rows[k], rsem[k]).wait()

    for k in range(6):
        pltpu.async_copy(src_hbm.at[wid, k], srcb[k], ssem[k])
    pltpu.async_copy(dst_hbm.at[wid], dst_v, sem_d)
    for k in range(3):
        _wait_src(k)
        pltpu.async_copy(x_hbm.at[srcb[k]], rows[k], rsem[k])
    pltpu.sync_copy(zeros_hbm.at[pl.ds(s * ROWS_PER_TILE, ROWS_PER_TILE)],
                    acc_sh.at[pl.ds(s * ROWS_PER_TILE, ROWS_PER_TILE)])
    pltpu.make_async_copy(dst_hbm.at[wid], dst_v, sem_d).wait()
    plsc.subcore_barrier()

    def _step(i, k, prefetch):
        _wait_rows(k % 3)
        pltpu.sync_copy(rows[k % 3], acc_sh.at[dst_v.at[i]], add=True)
        if prefetch:
            _wait_src((k + 3) % 6)
            pltpu.async_copy(x_hbm.at[srcb[(k + 3) % 6]], rows[k % 3],
                             rsem[k % 3])
            pltpu.async_copy(src_hbm.at[wid, i + 6], srcb[k % 6], ssem[k % 6])

    def body(j, carry):
        i = 6 * j
        for k in range(6):
            _step(i + k, k, True)
        return carry

    lax.fori_loop(0, NCHUNK // 6 - 1, body, 0)
    base = NCHUNK - 6
    for k in range(6):
        _wait_rows(k % 3)
        pltpu.sync_copy(rows[k % 3], acc_sh.at[dst_v.at[base + k]], add=True)
        if k < 3:
            _wait_src((k + 3) % 6)
            pltpu.async_copy(x_hbm.at[srcb[(k + 3) % 6]], rows[k % 3],
                             rsem[k % 3])
    plsc.subcore_barrier()

    pltpu.sync_copy(acc_sh.at[pl.ds(s * ROWS_PER_TILE, ROWS_PER_TILE)],
                    out_hbm.at[c, pl.ds(s * ROWS_PER_TILE, ROWS_PER_TILE)])


def _layer_body(parts_ref, x_ref, wrel_ref, brel_ref, wroot_ref, o_ref):
    agg = parts_ref[0] + parts_ref[1]
    acc = jnp.dot(agg, wrel_ref[...], preferred_element_type=jnp.float32)
    acc += jnp.dot(x_ref[...], wroot_ref[...], preferred_element_type=jnp.float32)
    acc += brel_ref[...]
    o_ref[...] = jnp.maximum(acc, 0.0)


def _layer_tc(parts, x, wrel, brel, wroot):
    return pl.pallas_call(
        _layer_body,
        grid=(NBLK,),
        in_specs=[
            pl.BlockSpec((NC, BR, D), lambda i: (0, i, 0)),
            pl.BlockSpec((BR, D), lambda i: (i, 0)),
            pl.BlockSpec((D, D), lambda i: (0, 0)),
            pl.BlockSpec((1, D), lambda i: (0, 0)),
            pl.BlockSpec((D, D), lambda i: (0, 0)),
        ],
        out_specs=pl.BlockSpec((BR, D), lambda i: (i, 0)),
        out_shape=jax.ShapeDtypeStruct((N, D), jnp.float32),
    )(parts, x, wrel, brel.reshape(1, D), wroot)


def _final_body(parts_ref, x_ref, wrel_ref, brel_ref, wroot_ref,
                wlin1_ref, blin1_ref, wlin2_ref, blin2_ref, batch_ref, o_ref):
    agg = parts_ref[0] + parts_ref[1]
    h = jnp.dot(agg, wrel_ref[...], preferred_element_type=jnp.float32)
    h += jnp.dot(x_ref[...], wroot_ref[...], preferred_element_type=jnp.float32)
    h += brel_ref[...]
    h = jnp.maximum(h, 0.0)
    h = jnp.maximum(
        jnp.dot(h, wlin1_ref[...], preferred_element_type=jnp.float32)
        + blin1_ref[...], 0.0)
    y = jnp.dot(h, wlin2_ref[...], preferred_element_type=jnp.float32)
    y += blin2_ref[...]
    seg = lax.broadcasted_iota(jnp.int32, (BR, G), 1)
    mask = (batch_ref[...] == seg).astype(jnp.float32)
    contrib = lax.dot_general(mask, y, (((0,), (0,)), ((), ())),
                              preferred_element_type=jnp.float32)

    @pl.when(pl.program_id(0) == 0)
    def _():
        o_ref[...] = jnp.zeros_like(o_ref)

    o_ref[...] += contrib


def _final_tc(parts, x, wrel, brel, wroot, wlin1, blin1, wlin2, blin2, batch):
    return pl.pallas_call(
        _final_body,
        grid=(NBLK,),
        in_specs=[
            pl.BlockSpec((NC, BR, D), lambda i: (0, i, 0)),
            pl.BlockSpec((BR, D), lambda i: (i, 0)),
            pl.BlockSpec((D, D), lambda i: (0, 0)),
            pl.BlockSpec((1, D), lambda i: (0, 0)),
            pl.BlockSpec((D, D), lambda i: (0, 0)),
            pl.BlockSpec((D, D), lambda i: (0, 0)),
            pl.BlockSpec((1, D), lambda i: (0, 0)),
            pl.BlockSpec((D, OUT), lambda i: (0, 0)),
            pl.BlockSpec((1, OUT), lambda i: (0, 0)),
            pl.BlockSpec((BR, 1), lambda i: (i, 0)),
        ],
        out_specs=pl.BlockSpec((G, OUT), lambda i: (0, 0)),
        out_shape=jax.ShapeDtypeStruct((G, OUT), jnp.float32),
    )(parts, x, wrel, brel.reshape(1, D), wroot,
      wlin1, blin1.reshape(1, D), wlin2, blin2.reshape(1, OUT),
      batch.reshape(N, 1))


def kernel(x, edge_index, batch,
           Wrel0, brel0, Wroot0,
           Wrel1, brel1, Wroot1,
           Wrel2, brel2, Wroot2,
           Wlin1, blin1, Wlin2, blin2):
    npad_e = EPAD - E
    src = jnp.concatenate(
        [edge_index[0], jnp.arange(npad_e, dtype=jnp.int32) % N]
    ).reshape(NW, NCHUNK, CHUNK)
    dst = jnp.concatenate(
        [edge_index[1], N + (jnp.arange(npad_e, dtype=jnp.int32) % (NPAD - N))]
    ).reshape(NW, NCHUNK, CHUNK)
    zeros = jnp.zeros((NPAD, D), jnp.float32)

    parts = _segsum_sc(x, src, dst, zeros)
    h = _layer_tc(parts, x, Wrel0, brel0, Wroot0)
    parts = _segsum_sc(h, src, dst, zeros)
    h = _layer_tc(parts, h, Wrel1, brel1, Wroot1)
    parts = _segsum_sc(h, src, dst, zeros)
    return _final_tc(parts, h, Wrel2, brel2, Wroot2,
                     Wlin1, blin1, Wlin2, blin2, batch)


# speedup vs baseline: 4.4904x; 1.0024x over previous
import functools

import jax
import jax.numpy as jnp
from jax import lax
from jax.experimental import pallas as pl
from jax.experimental.pallas import tpu as pltpu
from jax.experimental.pallas import tpu_sc as plsc

N = 10000
E = 320000
D = 128
G = 64
OUT = 10

NC = 2
NS = 16
NW = NC * NS
CHUNK = 80
NCHUNK = 126
EDGES_PER_TILE = NCHUNK * CHUNK
EPAD = NW * EDGES_PER_TILE
NPAD = 10240
ROWS_PER_TILE = NPAD // NS

BR = 2000
NBLK = N // BR


@functools.partial(
    pl.kernel,
    out_type=jax.ShapeDtypeStruct((NC, NPAD, D), jnp.float32),
    mesh=plsc.VectorSubcoreMesh(core_axis_name="c", subcore_axis_name="s"),
    scratch_types=[
        pltpu.VMEM((NCHUNK, CHUNK), jnp.int32),
        pltpu.VMEM((CHUNK,), jnp.int32),
        pltpu.VMEM((CHUNK,), jnp.int32),
        pltpu.VMEM((CHUNK,), jnp.int32),
        pltpu.VMEM((CHUNK,), jnp.int32),
        pltpu.VMEM((CHUNK,), jnp.int32),
        pltpu.VMEM((CHUNK,), jnp.int32),
        pltpu.VMEM((CHUNK, D), jnp.float32),
        pltpu.VMEM((CHUNK, D), jnp.float32),
        pltpu.VMEM((CHUNK, D), jnp.float32),
        pltpu.VMEM_SHARED((NPAD, D), jnp.float32),
        pltpu.SemaphoreType.DMA,
        pltpu.SemaphoreType.DMA,
        pltpu.SemaphoreType.DMA,
        pltpu.SemaphoreType.DMA,
        pltpu.SemaphoreType.DMA,
        pltpu.SemaphoreType.DMA,
        pltpu.SemaphoreType.DMA,
        pltpu.SemaphoreType.DMA,
        pltpu.SemaphoreType.DMA,
        pltpu.SemaphoreType.DMA,
        pltpu.SemaphoreType.DMA,
        pltpu.SemaphoreType.DMA,
        pltpu.SemaphoreType.DMA,
    ],
)
def _segsum_sc(x_hbm, src_hbm, dst_hbm, zeros_hbm, out_hbm,
               dst_v, sr0, sr1, sr2, sr3, sr4, sr5,
               rows_a, rows_b, rows_c, acc_sh,
               sem_a, sem_b, sem_c, ss0, ss1, ss2, ss3, ss4, ss5, sem_d,
               sc0, sc1, sc2):
    c = lax.axis_index("c")
    s = lax.axis_index("s")
    wid = c * NS + s
    srcb = (sr0, sr1, sr2, sr3, sr4, sr5)
    ssem = (ss0, ss1, ss2, ss3, ss4, ss5)
    rows = (rows_a, rows_b, rows_c)
    rsem = (sem_a, sem_b, sem_c)
    csem = (sc0, sc1, sc2)

    def _wait_src(k):
        pltpu.make_async_copy(src_hbm.at[wid, 0], srcb[k], ssem[k]).wait()

    def _wait_rows(k):
        pltpu.make_async_copy(x_hbm.at[sr0], rows[k], rsem[k]).wait()

    for k in range(6):
        pltpu.async_copy(src_hbm.at[wid, k], srcb[k], ssem[k])
    pltpu.async_copy(dst_hbm.at[wid], dst_v, sem_d)
    for k in range(3):
        _wait_src(k)
        pltpu.async_copy(x_hbm.at[srcb[k]], rows[k], rsem[k])
    pltpu.sync_copy(zeros_hbm.at[pl.ds(s * ROWS_PER_TILE, ROWS_PER_TILE)],
                    acc_sh.at[pl.ds(s * ROWS_PER_TILE, ROWS_PER_TILE)])
    pltpu.make_async_copy(dst_hbm.at[wid], dst_v, sem_d).wait()
    plsc.subcore_barrier()

    def _wait_scat(i, k):
        pltpu.make_async_copy(rows[k % 3], acc_sh.at[dst_v.at[i]],
                              csem[k % 3]).wait()

    def _step(i, k, prefetch):
        _wait_rows(k % 3)
        pltpu.async_copy(rows[k % 3], acc_sh.at[dst_v.at[i]], csem[k % 3],
                         add=True)
        if prefetch:
            _wait_src((k + 3) % 6)
            pltpu.async_copy(src_hbm.at[wid, i + 6], srcb[k % 6], ssem[k % 6])
            _wait_scat(i, k)
            pltpu.async_copy(x_hbm.at[srcb[(k + 3) % 6]], rows[k % 3],
                             rsem[k % 3])

    def body(j, carry):
        i = 6 * j
        for k in range(6):
            _step(i + k, k, True)
        return carry

    lax.fori_loop(0, NCHUNK // 6 - 1, body, 0)
    base = NCHUNK - 6
    for k in range(6):
        _wait_rows(k % 3)
        pltpu.async_copy(rows[k % 3], acc_sh.at[dst_v.at[base + k]],
                         csem[k % 3], add=True)
        if k < 3:
            _wait_src((k + 3) % 6)
            _wait_scat(base + k, k)
            pltpu.async_copy(x_hbm.at[srcb[(k + 3) % 6]], rows[k % 3],
                             rsem[k % 3])
    for k in range(3):
        _wait_scat(base + 3 + k, k + 3)
    plsc.subcore_barrier()

    pltpu.sync_copy(acc_sh.at[pl.ds(s * ROWS_PER_TILE, ROWS_PER_TILE)],
                    out_hbm.at[c, pl.ds(s * ROWS_PER_TILE, ROWS_PER_TILE)])


def _layer_body(parts_ref, x_ref, wrel_ref, brel_ref, wroot_ref, o_ref):
    agg = parts_ref[0] + parts_ref[1]
    acc = jnp.dot(agg, wrel_ref[...], preferred_element_type=jnp.float32)
    acc += jnp.dot(x_ref[...], wroot_ref[...], preferred_element_type=jnp.float32)
    acc += brel_ref[...]
    o_ref[...] = jnp.maximum(acc, 0.0)


def _layer_tc(parts, x, wrel, brel, wroot):
    return pl.pallas_call(
        _layer_body,
        grid=(NBLK,),
        in_specs=[
            pl.BlockSpec((NC, BR, D), lambda i: (0, i, 0)),
            pl.BlockSpec((BR, D), lambda i: (i, 0)),
            pl.BlockSpec((D, D), lambda i: (0, 0)),
            pl.BlockSpec((1, D), lambda i: (0, 0)),
            pl.BlockSpec((D, D), lambda i: (0, 0)),
        ],
        out_specs=pl.BlockSpec((BR, D), lambda i: (i, 0)),
        out_shape=jax.ShapeDtypeStruct((N, D), jnp.float32),
    )(parts, x, wrel, brel.reshape(1, D), wroot)


def _final_body(parts_ref, x_ref, wrel_ref, brel_ref, wroot_ref,
                wlin1_ref, blin1_ref, wlin2_ref, blin2_ref, batch_ref, o_ref):
    agg = parts_ref[0] + parts_ref[1]
    h = jnp.dot(agg, wrel_ref[...], preferred_element_type=jnp.float32)
    h += jnp.dot(x_ref[...], wroot_ref[...], preferred_element_type=jnp.float32)
    h += brel_ref[...]
    h = jnp.maximum(h, 0.0)
    h = jnp.maximum(
        jnp.dot(h, wlin1_ref[...], preferred_element_type=jnp.float32)
        + blin1_ref[...], 0.0)
    y = jnp.dot(h, wlin2_ref[...], preferred_element_type=jnp.float32)
    y += blin2_ref[...]
    seg = lax.broadcasted_iota(jnp.int32, (BR, G), 1)
    mask = (batch_ref[...] == seg).astype(jnp.float32)
    contrib = lax.dot_general(mask, y, (((0,), (0,)), ((), ())),
                              preferred_element_type=jnp.float32)

    @pl.when(pl.program_id(0) == 0)
    def _():
        o_ref[...] = jnp.zeros_like(o_ref)

    o_ref[...] += contrib


def _final_tc(parts, x, wrel, brel, wroot, wlin1, blin1, wlin2, blin2, batch):
    return pl.pallas_call(
        _final_body,
        grid=(NBLK,),
        in_specs=[
            pl.BlockSpec((NC, BR, D), lambda i: (0, i, 0)),
            pl.BlockSpec((BR, D), lambda i: (i, 0)),
            pl.BlockSpec((D, D), lambda i: (0, 0)),
            pl.BlockSpec((1, D), lambda i: (0, 0)),
            pl.BlockSpec((D, D), lambda i: (0, 0)),
            pl.BlockSpec((D, D), lambda i: (0, 0)),
            pl.BlockSpec((1, D), lambda i: (0, 0)),
            pl.BlockSpec((D, OUT), lambda i: (0, 0)),
            pl.BlockSpec((1, OUT), lambda i: (0, 0)),
            pl.BlockSpec((BR, 1), lambda i: (i, 0)),
        ],
        out_specs=pl.BlockSpec((G, OUT), lambda i: (0, 0)),
        out_shape=jax.ShapeDtypeStruct((G, OUT), jnp.float32),
    )(parts, x, wrel, brel.reshape(1, D), wroot,
      wlin1, blin1.reshape(1, D), wlin2, blin2.reshape(1, OUT),
      batch.reshape(N, 1))


def kernel(x, edge_index, batch,
           Wrel0, brel0, Wroot0,
           Wrel1, brel1, Wroot1,
           Wrel2, brel2, Wroot2,
           Wlin1, blin1, Wlin2, blin2):
    npad_e = EPAD - E
    src = jnp.concatenate(
        [edge_index[0], jnp.arange(npad_e, dtype=jnp.int32) % N]
    ).reshape(NW, NCHUNK, CHUNK)
    dst = jnp.concatenate(
        [edge_index[1], N + (jnp.arange(npad_e, dtype=jnp.int32) % (NPAD - N))]
    ).reshape(NW, NCHUNK, CHUNK)
    zeros = jnp.zeros((NPAD, D), jnp.float32)

    parts = _segsum_sc(x, src, dst, zeros)
    h = _layer_tc(parts, x, Wrel0, brel0, Wroot0)
    parts = _segsum_sc(h, src, dst, zeros)
    h = _layer_tc(parts, h, Wrel1, brel1, Wroot1)
    parts = _segsum_sc(h, src, dst, zeros)
    return _final_tc(parts, h, Wrel2, brel2, Wroot2,
                     Wlin1, blin1, Wlin2, blin2, batch)
